# Initial kernel scaffold; baseline (speedup 1.0000x reference)
import jax
import jax.numpy as jnp
from jax.experimental import pallas as pl


def kernel(word_indices, table):
    raise NotImplementedError("write your pallas kernel here")


# speedup vs baseline: 1.4633x; 1.4633x over previous
import functools

import jax
import jax.numpy as jnp
from jax import lax
from jax.experimental import pallas as pl
from jax.experimental.pallas import tpu as pltpu
from jax.experimental.pallas import tpu_sc as plsc

VOCAB = 1000000
EMBED_DIM = 32
BATCH = 4096
HIST = 200
TOTAL = BATCH * HIST

NUM_CORES = 2
NUM_SUBCORES = 16
NW = NUM_CORES * NUM_SUBCORES
B_PER_W = TOTAL // NW
CHUNK = 1024
N_CHUNKS = B_PER_W // CHUNK


def _make_gather_kernel():
    mesh = plsc.VectorSubcoreMesh(core_axis_name="c", subcore_axis_name="s")

    @functools.partial(
        pl.kernel,
        mesh=mesh,
        out_type=jax.ShapeDtypeStruct((TOTAL, EMBED_DIM), jnp.float32),
        scratch_types=[
            pltpu.VMEM((CHUNK,), jnp.int32),
            pltpu.VMEM((CHUNK, EMBED_DIM), jnp.float32),
            pltpu.SemaphoreType.DMA,
        ],
        compiler_params=pltpu.CompilerParams(use_tc_tiling_on_sc=False),
    )
    def gather_kernel(table_hbm, idx_hbm, out_hbm, idx_v, rows_v, sem):
        wid = lax.axis_index("s") * NUM_CORES + lax.axis_index("c")
        base = wid * B_PER_W

        def body(i, carry):
            off = base + i * CHUNK
            pltpu.sync_copy(idx_hbm.at[pl.ds(off, CHUNK)], idx_v)
            pltpu.async_copy(table_hbm.at[idx_v], rows_v, sem).wait()
            pltpu.sync_copy(rows_v, out_hbm.at[pl.ds(off, CHUNK)])
            return carry

        lax.fori_loop(0, N_CHUNKS, body, 0)

    return gather_kernel


_gather = _make_gather_kernel()


@jax.jit
def kernel(word_indices, table):
    idx_flat = word_indices.reshape(TOTAL).astype(jnp.int32)
    out = _gather(table, idx_flat)
    return out.reshape(BATCH, HIST, EMBED_DIM)


# speedup vs baseline: 1.4944x; 1.0213x over previous
import functools

import jax
import jax.numpy as jnp
from jax import lax
from jax.experimental import pallas as pl
from jax.experimental.pallas import tpu as pltpu
from jax.experimental.pallas import tpu_sc as plsc

VOCAB = 1000000
EMBED_DIM = 32
BATCH = 4096
HIST = 200
TOTAL = BATCH * HIST

NUM_CORES = 2
NUM_SUBCORES = 16
NW = NUM_CORES * NUM_SUBCORES
B_PER_W = TOTAL // NW
CHUNK = 1600
N_CHUNKS = B_PER_W // CHUNK
NBUF = 2


def _make_gather_kernel():
    mesh = plsc.VectorSubcoreMesh(core_axis_name="c", subcore_axis_name="s")

    scratch = []
    for _ in range(NBUF):
        scratch.append(pltpu.VMEM((CHUNK,), jnp.int32))
        scratch.append(pltpu.VMEM((CHUNK, EMBED_DIM), jnp.float32))
        scratch.append(pltpu.SemaphoreType.DMA)
        scratch.append(pltpu.SemaphoreType.DMA)

    @functools.partial(
        pl.kernel,
        mesh=mesh,
        out_type=jax.ShapeDtypeStruct((TOTAL, EMBED_DIM), jnp.float32),
        scratch_types=scratch,
        compiler_params=pltpu.CompilerParams(use_tc_tiling_on_sc=False),
    )
    def gather_kernel(table_hbm, idx_hbm, out_hbm, *bufs):
        idx_v = [bufs[4 * b + 0] for b in range(NBUF)]
        rows_v = [bufs[4 * b + 1] for b in range(NBUF)]
        sem_g = [bufs[4 * b + 2] for b in range(NBUF)]
        sem_w = [bufs[4 * b + 3] for b in range(NBUF)]

        wid = lax.axis_index("s") * NUM_CORES + lax.axis_index("c")
        base = wid * B_PER_W

        gathers = [None] * NBUF
        writebacks = [None] * NBUF
        for i in range(N_CHUNKS):
            b = i % NBUF
            if writebacks[b] is not None:
                writebacks[b].wait()
            off = base + i * CHUNK
            pltpu.sync_copy(idx_hbm.at[pl.ds(off, CHUNK)], idx_v[b])
            gathers[b] = pltpu.async_copy(
                table_hbm.at[idx_v[b]], rows_v[b], sem_g[b])
            if i > 0:
                pb = (i - 1) % NBUF
                gathers[pb].wait()
                poff = base + (i - 1) * CHUNK
                writebacks[pb] = pltpu.async_copy(
                    rows_v[pb], out_hbm.at[pl.ds(poff, CHUNK)], sem_w[pb])
        lb = (N_CHUNKS - 1) % NBUF
        gathers[lb].wait()
        loff = base + (N_CHUNKS - 1) * CHUNK
        writebacks[lb] = pltpu.async_copy(
            rows_v[lb], out_hbm.at[pl.ds(loff, CHUNK)], sem_w[lb])
        for b in range(NBUF):
            if writebacks[b] is not None:
                writebacks[b].wait()

    return gather_kernel


_gather = _make_gather_kernel()


@jax.jit
def kernel(word_indices, table):
    idx_flat = word_indices.reshape(TOTAL).astype(jnp.int32)
    out = _gather(table, idx_flat)
    return out.reshape(BATCH, HIST, EMBED_DIM)
